# baseline (speedup 1.0000x reference)
import functools

import jax
import jax.numpy as jnp
from jax import lax
from jax.experimental import pallas as pl
from jax.experimental.pallas import tpu as pltpu
from jax.experimental.pallas import tpu_sc as plsc

N_IN_ROWS = 50000
N_OUT_ROWS = 25000
N_ENTRIES = 100000
DIM = 256

NC = 2
NS = 16
NW = NC * NS
LANES = 16

NBUF = 4
CH = 24
NE = CH * 4
NCHUNK = N_OUT_ROWS // CH
TAIL_CH = N_OUT_ROWS - NCHUNK * CH
TAIL_NE = TAIL_CH * 4
MAXC = -(-NCHUNK // NW)
NBIG = NCHUNK - (MAXC - 1) * NW
NVREG = DIM // LANES


def _dyn_gather(vec, idx):
    dnums = lax.GatherDimensionNumbers(
        offset_dims=(), collapsed_slice_dims=(0,), start_index_map=(0,))
    return lax.gather(vec, idx[:, None], dnums, (1,),
                      mode=lax.GatherScatterMode.PROMISE_IN_BOUNDS)


def _splat(vec, lane):
    return _dyn_gather(vec, jnp.full((LANES,), lane, jnp.int32))


def _rows_block(vals_v, voff, gath_v, out_v, n_rows):
    lanes = lax.iota(jnp.int32, LANES)
    x1 = lanes ^ 1
    x2 = lanes ^ 2

    @plsc.parallel_loop(0, n_rows, unroll=4)
    def row_body(i):
        b = 4 * i
        vv = vals_v[pl.ds(voff + b, LANES)]
        s1 = vv + _dyn_gather(vv, x1)
        s4 = s1 + _dyn_gather(s1, x2)
        av = vv / s4
        a0 = _splat(av, 0)
        a1 = _splat(av, 1)
        a2 = _splat(av, 2)
        a3 = _splat(av, 3)
        for d in range(NVREG):
            sl = pl.ds(d * LANES, LANES)
            acc = ((a0 * gath_v[b, sl] + a1 * gath_v[b + 1, sl])
                   + (a2 * gath_v[b + 2, sl] + a3 * gath_v[b + 3, sl]))
            out_v[i, sl] = acc


def _sc_body(input_hbm, cols_hbm, vals_hbm, out_hbm,
             colsall_v, valsall_v,
             gath0, gath1, gath2, gath3, out0, out1, out2, out3,
             cols_t, vals_t,
             sem_s, sg0, sg1, sg2, sg3, so0, so1, so2, so3):
    wid = lax.axis_index("s") * NC + lax.axis_index("c")
    base = wid * MAXC - jnp.maximum(wid - NBIG, 0)
    limit = jnp.where(wid < NBIG, MAXC, MAXC - 1)

    def stage(nchunks, op):
        e0 = base * NE
        n = nchunks * NE
        c = pltpu.make_async_copy(cols_hbm.at[pl.ds(e0, n)],
                                  colsall_v.at[pl.ds(0, n)], sem_s)
        v = pltpu.make_async_copy(vals_hbm.at[pl.ds(e0, n)],
                                  valsall_v.at[pl.ds(0, n)], sem_s)
        getattr(c, op)()
        getattr(v, op)()

    @pl.when(wid < NBIG)
    def _():
        stage(MAXC, "start")
        stage(MAXC, "wait")

    @pl.when(wid >= NBIG)
    def _():
        stage(MAXC - 1, "start")
        stage(MAXC - 1, "wait")

    bufs = ((gath0, out0, sg0, so0), (gath1, out1, sg1, so1),
            (gath2, out2, sg2, so2), (gath3, out3, sg3, so3))

    def gather_desc(k, gath_b, sem_b):
        return pltpu.make_async_copy(
            input_hbm.at[colsall_v.at[pl.ds(k * NE, NE)]], gath_b, sem_b)

    for p in range(NBUF):
        gather_desc(p, bufs[p][0], bufs[p][2]).start()

    def jbody(j, _):
        for parity in range(NBUF):
            gath_b, out_b, sg_b, so_b = bufs[parity]
            k = NBUF * j + parity
            c = base + k

            @pl.when(k < limit)
            def _():
                gather_desc(k, gath_b, sg_b).wait()

                @pl.when(k >= NBUF)
                def _():
                    pltpu.make_async_copy(
                        out_b, out_hbm.at[pl.ds((c - NBUF) * CH, CH)],
                        so_b).wait()

                _rows_block(valsall_v, k * NE, gath_b, out_b, CH)
                pltpu.make_async_copy(
                    out_b, out_hbm.at[pl.ds(c * CH, CH)], so_b).start()

                @pl.when(k + NBUF < limit)
                def _():
                    gather_desc(k + NBUF, gath_b, sg_b).start()

        return 0

    lax.fori_loop(0, -(-MAXC // NBUF), jbody, 0)

    for parity in range(NBUF):
        _, out_b, _, so_b = bufs[parity]
        pltpu.make_async_copy(out_b, out_hbm.at[pl.ds(0, CH)], so_b).wait()

    @pl.when(wid == 0)
    def _():
        e0 = NCHUNK * NE
        pltpu.sync_copy(cols_hbm.at[pl.ds(e0, TAIL_NE)], cols_t)
        pltpu.sync_copy(vals_hbm.at[pl.ds(e0, TAIL_NE)],
                        vals_t.at[pl.ds(0, TAIL_NE)])
        gath_t = gath0.at[pl.ds(0, TAIL_NE)]
        out_t = out0.at[pl.ds(0, TAIL_CH)]
        pltpu.make_async_copy(input_hbm.at[cols_t], gath_t, sem_s).start()
        pltpu.make_async_copy(input_hbm.at[cols_t], gath_t, sem_s).wait()
        _rows_block(vals_t, 0, gath0, out0, TAIL_CH)
        pltpu.sync_copy(out_t, out_hbm.at[pl.ds(NCHUNK * CH, TAIL_CH)])


@jax.jit
def _mesh_pool(input, cols_i32, vals):
    mesh = plsc.VectorSubcoreMesh(core_axis_name="c", subcore_axis_name="s")
    f = functools.partial(
        pl.kernel,
        mesh=mesh,
        out_type=jax.ShapeDtypeStruct((N_OUT_ROWS, DIM), jnp.float32),
        scratch_types=[
            pltpu.VMEM((MAXC * NE,), jnp.int32),
            pltpu.VMEM((MAXC * NE + LANES,), jnp.float32),
            pltpu.VMEM((NE, DIM), jnp.float32),
            pltpu.VMEM((NE, DIM), jnp.float32),
            pltpu.VMEM((NE, DIM), jnp.float32),
            pltpu.VMEM((NE, DIM), jnp.float32),
            pltpu.VMEM((CH, DIM), jnp.float32),
            pltpu.VMEM((CH, DIM), jnp.float32),
            pltpu.VMEM((CH, DIM), jnp.float32),
            pltpu.VMEM((CH, DIM), jnp.float32),
            pltpu.VMEM((TAIL_NE,), jnp.int32),
            pltpu.VMEM((TAIL_NE + LANES,), jnp.float32),
        ] + [pltpu.SemaphoreType.DMA] * 9,
    )(_sc_body)
    return f(input, cols_i32, vals)


def kernel(input, rows, cols, vals):
    del rows
    return _mesh_pool(input, cols.astype(jnp.int32), vals.astype(jnp.float32))


# speedup vs baseline: 1.0427x; 1.0427x over previous
import functools

import jax
import jax.numpy as jnp
from jax import lax
from jax.experimental import pallas as pl
from jax.experimental.pallas import tpu as pltpu
from jax.experimental.pallas import tpu_sc as plsc

N_IN_ROWS = 50000
N_OUT_ROWS = 25000
N_ENTRIES = 100000
DIM = 256

NC = 2
NS = 16
NW = NC * NS
LANES = 16

NBUF = 3
CH = 32
NE = CH * 4
NCHUNK = N_OUT_ROWS // CH
TAIL_CH = N_OUT_ROWS - NCHUNK * CH
TAIL_NE = TAIL_CH * 4
MAXC = -(-NCHUNK // NW)
NBIG = NCHUNK - (MAXC - 1) * NW
NVREG = DIM // LANES


def _dyn_gather(vec, idx):
    dnums = lax.GatherDimensionNumbers(
        offset_dims=(), collapsed_slice_dims=(0,), start_index_map=(0,))
    return lax.gather(vec, idx[:, None], dnums, (1,),
                      mode=lax.GatherScatterMode.PROMISE_IN_BOUNDS)


def _splat(vec, lane):
    return _dyn_gather(vec, jnp.full((LANES,), lane, jnp.int32))


def _rows_block(vals_v, voff, gath_v, out_v, n_rows):
    lanes = lax.iota(jnp.int32, LANES)
    x1 = lanes ^ 1
    x2 = lanes ^ 2

    @plsc.parallel_loop(0, n_rows, unroll=4)
    def row_body(i):
        b = 4 * i
        vv = vals_v[pl.ds(voff + b, LANES)]
        s1 = vv + _dyn_gather(vv, x1)
        s4 = s1 + _dyn_gather(s1, x2)
        av = vv / s4
        a0 = _splat(av, 0)
        a1 = _splat(av, 1)
        a2 = _splat(av, 2)
        a3 = _splat(av, 3)
        for d in range(NVREG):
            sl = pl.ds(d * LANES, LANES)
            acc = ((a0 * gath_v[b, sl] + a1 * gath_v[b + 1, sl])
                   + (a2 * gath_v[b + 2, sl] + a3 * gath_v[b + 3, sl]))
            out_v[i, sl] = acc


def _sc_body(input_hbm, cols_hbm, vals_hbm, out_hbm,
             colsall_v, valsall_v,
             gath0, gath1, gath2, out0, out1, out2,
             cols_t, vals_t,
             sem_s, sg0, sg1, sg2, so0, so1, so2):
    wid = lax.axis_index("s") * NC + lax.axis_index("c")
    base = wid * MAXC - jnp.maximum(wid - NBIG, 0)
    limit = jnp.where(wid < NBIG, MAXC, MAXC - 1)

    def stage(nchunks, op):
        e0 = base * NE
        n = nchunks * NE
        c = pltpu.make_async_copy(cols_hbm.at[pl.ds(e0, n)],
                                  colsall_v.at[pl.ds(0, n)], sem_s)
        v = pltpu.make_async_copy(vals_hbm.at[pl.ds(e0, n)],
                                  valsall_v.at[pl.ds(0, n)], sem_s)
        getattr(c, op)()
        getattr(v, op)()

    @pl.when(wid < NBIG)
    def _():
        stage(MAXC, "start")
        stage(MAXC, "wait")

    @pl.when(wid >= NBIG)
    def _():
        stage(MAXC - 1, "start")
        stage(MAXC - 1, "wait")

    bufs = ((gath0, out0, sg0, so0), (gath1, out1, sg1, so1),
            (gath2, out2, sg2, so2))

    def gather_desc(k, gath_b, sem_b):
        return pltpu.make_async_copy(
            input_hbm.at[colsall_v.at[pl.ds(k * NE, NE)]], gath_b, sem_b)

    for p in range(NBUF):
        gather_desc(p, bufs[p][0], bufs[p][2]).start()

    def jbody(j, _):
        for parity in range(NBUF):
            gath_b, out_b, sg_b, so_b = bufs[parity]
            k = NBUF * j + parity
            c = base + k

            @pl.when(k < limit)
            def _():
                gather_desc(k, gath_b, sg_b).wait()

                @pl.when(k >= NBUF)
                def _():
                    pltpu.make_async_copy(
                        out_b, out_hbm.at[pl.ds((c - NBUF) * CH, CH)],
                        so_b).wait()

                _rows_block(valsall_v, k * NE, gath_b, out_b, CH)
                pltpu.make_async_copy(
                    out_b, out_hbm.at[pl.ds(c * CH, CH)], so_b).start()

                @pl.when(k + NBUF < limit)
                def _():
                    gather_desc(k + NBUF, gath_b, sg_b).start()

        return 0

    lax.fori_loop(0, -(-MAXC // NBUF), jbody, 0)

    for parity in range(NBUF):
        _, out_b, _, so_b = bufs[parity]
        pltpu.make_async_copy(out_b, out_hbm.at[pl.ds(0, CH)], so_b).wait()

    @pl.when(wid == 0)
    def _():
        e0 = NCHUNK * NE
        pltpu.sync_copy(cols_hbm.at[pl.ds(e0, TAIL_NE)], cols_t)
        pltpu.sync_copy(vals_hbm.at[pl.ds(e0, TAIL_NE)],
                        vals_t.at[pl.ds(0, TAIL_NE)])
        gath_t = gath0.at[pl.ds(0, TAIL_NE)]
        out_t = out0.at[pl.ds(0, TAIL_CH)]
        pltpu.make_async_copy(input_hbm.at[cols_t], gath_t, sem_s).start()
        pltpu.make_async_copy(input_hbm.at[cols_t], gath_t, sem_s).wait()
        _rows_block(vals_t, 0, gath0, out0, TAIL_CH)
        pltpu.sync_copy(out_t, out_hbm.at[pl.ds(NCHUNK * CH, TAIL_CH)])


@jax.jit
def _mesh_pool(input, cols_i32, vals):
    mesh = plsc.VectorSubcoreMesh(core_axis_name="c", subcore_axis_name="s")
    f = functools.partial(
        pl.kernel,
        mesh=mesh,
        out_type=jax.ShapeDtypeStruct((N_OUT_ROWS, DIM), jnp.float32),
        scratch_types=[
            pltpu.VMEM((MAXC * NE,), jnp.int32),
            pltpu.VMEM((MAXC * NE + LANES,), jnp.float32),
            pltpu.VMEM((NE, DIM), jnp.float32),
            pltpu.VMEM((NE, DIM), jnp.float32),
            pltpu.VMEM((NE, DIM), jnp.float32),
            pltpu.VMEM((CH, DIM), jnp.float32),
            pltpu.VMEM((CH, DIM), jnp.float32),
            pltpu.VMEM((CH, DIM), jnp.float32),
            pltpu.VMEM((TAIL_NE,), jnp.int32),
            pltpu.VMEM((TAIL_NE + LANES,), jnp.float32),
        ] + [pltpu.SemaphoreType.DMA] * 7,
    )(_sc_body)
    return f(input, cols_i32, vals)


def kernel(input, rows, cols, vals):
    del rows
    return _mesh_pool(input, cols.astype(jnp.int32), vals.astype(jnp.float32))
